# baseline (speedup 1.0000x reference)
import functools
import math

import jax
import jax.numpy as jnp
from jax import lax
from jax.experimental import pallas as pl
from jax.experimental.pallas import tpu as pltpu
from jax.experimental.pallas import tpu_sc as plsc

_TEMP = 0.05
_CHUNK = 512
_NSPLIT = 4
_SUB = _CHUNK // _NSPLIT
_LOG2E = math.log2(math.e)


def _make_sc_gather(b, d):
    info = plsc.get_sparse_core_info()
    nc, ns = info.num_cores, info.num_subcores
    nw = nc * ns
    assert b % (8 * nw) == 0 and d % 128 == 0
    b_per_w = b // nw
    mesh = plsc.VectorSubcoreMesh(core_axis_name="c", subcore_axis_name="s")

    @functools.partial(
        pl.kernel, mesh=mesh,
        out_type=jax.ShapeDtypeStruct((b, d), jnp.float32),
        scratch_types=[
            pltpu.VMEM((b_per_w,), jnp.int32),
            pltpu.VMEM((b_per_w, d), jnp.float32),
            pltpu.SemaphoreType.DMA,
        ],
    )
    def gather_k(table_hbm, idx_hbm, out_hbm, idx_v, rows_v, sem):
        wid = lax.axis_index("s") * nc + lax.axis_index("c")
        base = wid * b_per_w
        pltpu.sync_copy(idx_hbm.at[pl.ds(base, b_per_w)], idx_v)
        pltpu.async_copy(table_hbm.at[idx_v], rows_v, sem).wait()
        pltpu.sync_copy(rows_v, out_hbm.at[pl.ds(base, b_per_w)])

    return gather_k


def _ce_kernel(n_valid, n_rows, x_ref, f_ref, g_ref, out_ref,
               xn_ref, s_ref):
    c = pl.program_id(0)
    nc = pl.num_programs(0)

    @pl.when(c == 0)
    def _init():
        x = x_ref[...]
        norm = jnp.sqrt(jnp.sum(x * x, axis=1, keepdims=True))
        scale = _LOG2E / (jnp.maximum(norm, 1e-12) * _TEMP)
        xn_ref[...] = (x * scale).astype(jnp.bfloat16)
        s_ref[...] = jnp.zeros_like(s_ref)

    xn = xn_ref[...]
    evs = []
    for k in range(_NSPLIT):
        m = jax.lax.dot_general(
            xn, f_ref[k * _SUB:(k + 1) * _SUB, :], (((1,), (1,)), ((), ())),
            preferred_element_type=jnp.float32)
        evs.append(jnp.exp2(m))

    @pl.when(c < nc - 1)
    def _mid():
        s_ref[...] += (evs[0] + evs[1]) + (evs[2] + evs[3])

    @pl.when(c == nc - 1)
    def _tail():
        lane = jax.lax.broadcasted_iota(jnp.int32, evs[0].shape, 1)
        acc = jnp.zeros_like(evs[0])
        for k in range(_NSPLIT):
            acc += jnp.where(lane < n_valid - c * _CHUNK - k * _SUB,
                             evs[k], 0.0)
        s_ref[...] += acc

    @pl.when(c == nc - 1)
    def _fin():
        lse = jnp.log(jnp.sum(s_ref[...], axis=1, keepdims=True))
        x = x_ref[...]
        norm = jnp.sqrt(jnp.sum(x * x, axis=1, keepdims=True))
        xh = x / (jnp.maximum(norm, 1e-12) * _TEMP)
        tl = jnp.sum(xh * g_ref[:, 0:64])
        out_ref[...] = ((jnp.sum(lse) - tl) * (1.0 / n_rows)).reshape(1, 1)


@jax.jit
def kernel(inputs, targets, cameras, features):
    b, d = inputs.shape
    n = features.shape[0]
    nc = pl.cdiv(n, _CHUNK)
    n_pad = nc * _CHUNK
    fpad = jnp.pad(features.astype(jnp.bfloat16), ((0, n_pad - n), (0, 0)))
    f128 = jnp.pad(features, ((0, 0), (0, 128 - d)))
    g = _make_sc_gather(b, 128)(f128, targets.astype(jnp.int32))
    out = pl.pallas_call(
        functools.partial(_ce_kernel, n, b),
        grid=(nc,),
        in_specs=[
            pl.BlockSpec((b, d), lambda i: (0, 0)),
            pl.BlockSpec((_CHUNK, d), lambda i: (i, 0)),
            pl.BlockSpec((b, 128), lambda i: (0, 0)),
        ],
        out_specs=pl.BlockSpec((1, 1), lambda i: (0, 0)),
        out_shape=jax.ShapeDtypeStruct((1, 1), jnp.float32),
        scratch_shapes=[
            pltpu.VMEM((b, d), jnp.bfloat16),
            pltpu.VMEM((b, 128), jnp.float32),
        ],
        compiler_params=pltpu.CompilerParams(
            dimension_semantics=("arbitrary",)),
    )(inputs, fpad, g)
    return out[0, 0]


# speedup vs baseline: 1.0705x; 1.0705x over previous
import functools
import math

import jax
import jax.numpy as jnp
from jax import lax
from jax.experimental import pallas as pl
from jax.experimental.pallas import tpu as pltpu
from jax.experimental.pallas import tpu_sc as plsc

_TEMP = 0.05
_CHUNK = 1024
_NSPLIT = 8
_SUB = _CHUNK // _NSPLIT
_LOG2E = math.log2(math.e)


def _make_sc_gather(b, d):
    info = plsc.get_sparse_core_info()
    nc, ns = info.num_cores, info.num_subcores
    nw = nc * ns
    assert b % (8 * nw) == 0 and d % 128 == 0
    b_per_w = b // nw
    mesh = plsc.VectorSubcoreMesh(core_axis_name="c", subcore_axis_name="s")

    @functools.partial(
        pl.kernel, mesh=mesh,
        out_type=jax.ShapeDtypeStruct((b, d), jnp.float32),
        scratch_types=[
            pltpu.VMEM((b_per_w,), jnp.int32),
            pltpu.VMEM((b_per_w, d), jnp.float32),
            pltpu.SemaphoreType.DMA,
        ],
    )
    def gather_k(table_hbm, idx_hbm, out_hbm, idx_v, rows_v, sem):
        wid = lax.axis_index("s") * nc + lax.axis_index("c")
        base = wid * b_per_w
        pltpu.sync_copy(idx_hbm.at[pl.ds(base, b_per_w)], idx_v)
        pltpu.async_copy(table_hbm.at[idx_v], rows_v, sem).wait()
        pltpu.sync_copy(rows_v, out_hbm.at[pl.ds(base, b_per_w)])

    return gather_k


def _ce_kernel(n_valid, n_rows, x_ref, f_ref, g_ref, out_ref,
               xn_ref, s_ref):
    c = pl.program_id(0)
    nc = pl.num_programs(0)

    @pl.when(c == 0)
    def _init():
        x = x_ref[...]
        norm = jnp.sqrt(jnp.sum(x * x, axis=1, keepdims=True))
        scale = _LOG2E / (jnp.maximum(norm, 1e-12) * _TEMP)
        xn_ref[...] = (x * scale).astype(jnp.bfloat16)
        s_ref[...] = jnp.zeros_like(s_ref)

    xn = xn_ref[...]
    evs = []
    for k in range(_NSPLIT):
        m = jax.lax.dot_general(
            xn, f_ref[k * _SUB:(k + 1) * _SUB, :], (((1,), (1,)), ((), ())),
            preferred_element_type=jnp.float32)
        evs.append(jnp.exp2(m))

    @pl.when(c < nc - 1)
    def _mid():
        s_ref[...] += (((evs[0] + evs[1]) + (evs[2] + evs[3]))
                       + ((evs[4] + evs[5]) + (evs[6] + evs[7])))

    @pl.when(c == nc - 1)
    def _tail():
        lane = jax.lax.broadcasted_iota(jnp.int32, evs[0].shape, 1)
        acc = jnp.zeros_like(evs[0])
        for k in range(_NSPLIT):
            acc += jnp.where(lane < n_valid - c * _CHUNK - k * _SUB,
                             evs[k], 0.0)
        s_ref[...] += acc

    @pl.when(c == nc - 1)
    def _fin():
        lse = jnp.log(jnp.sum(s_ref[...], axis=1, keepdims=True))
        x = x_ref[...]
        norm = jnp.sqrt(jnp.sum(x * x, axis=1, keepdims=True))
        xh = x / (jnp.maximum(norm, 1e-12) * _TEMP)
        tl = jnp.sum(xh * g_ref[:, 0:64])
        out_ref[...] = ((jnp.sum(lse) - tl) * (1.0 / n_rows)).reshape(1, 1)


@jax.jit
def kernel(inputs, targets, cameras, features):
    b, d = inputs.shape
    n = features.shape[0]
    nc = pl.cdiv(n, _CHUNK)
    n_pad = nc * _CHUNK
    fpad = jnp.pad(features.astype(jnp.bfloat16), ((0, n_pad - n), (0, 0)))
    f128 = jnp.pad(features, ((0, 0), (0, 128 - d)))
    g = _make_sc_gather(b, 128)(f128, targets.astype(jnp.int32))
    out = pl.pallas_call(
        functools.partial(_ce_kernel, n, b),
        grid=(nc,),
        in_specs=[
            pl.BlockSpec((b, d), lambda i: (0, 0)),
            pl.BlockSpec((_CHUNK, d), lambda i: (i, 0)),
            pl.BlockSpec((b, 128), lambda i: (0, 0)),
        ],
        out_specs=pl.BlockSpec((1, 1), lambda i: (0, 0)),
        out_shape=jax.ShapeDtypeStruct((1, 1), jnp.float32),
        scratch_shapes=[
            pltpu.VMEM((b, d), jnp.bfloat16),
            pltpu.VMEM((b, 128), jnp.float32),
        ],
        compiler_params=pltpu.CompilerParams(
            dimension_semantics=("arbitrary",)),
    )(inputs, fpad, g)
    return out[0, 0]


# speedup vs baseline: 1.1069x; 1.0340x over previous
import functools
import math

import jax
import jax.numpy as jnp
from jax import lax
from jax.experimental import pallas as pl
from jax.experimental.pallas import tpu as pltpu
from jax.experimental.pallas import tpu_sc as plsc

_TEMP = 0.05
_CHUNK = 2048
_NSPLIT = 16
_SUB = _CHUNK // _NSPLIT
_LOG2E = math.log2(math.e)


def _make_sc_gather(b, d):
    info = plsc.get_sparse_core_info()
    nc, ns = info.num_cores, info.num_subcores
    nw = nc * ns
    assert b % (8 * nw) == 0 and d % 128 == 0
    b_per_w = b // nw
    mesh = plsc.VectorSubcoreMesh(core_axis_name="c", subcore_axis_name="s")

    @functools.partial(
        pl.kernel, mesh=mesh,
        out_type=jax.ShapeDtypeStruct((b, d), jnp.float32),
        scratch_types=[
            pltpu.VMEM((b_per_w,), jnp.int32),
            pltpu.VMEM((b_per_w, d), jnp.float32),
            pltpu.SemaphoreType.DMA,
        ],
    )
    def gather_k(table_hbm, idx_hbm, out_hbm, idx_v, rows_v, sem):
        wid = lax.axis_index("s") * nc + lax.axis_index("c")
        base = wid * b_per_w
        pltpu.sync_copy(idx_hbm.at[pl.ds(base, b_per_w)], idx_v)
        pltpu.async_copy(table_hbm.at[idx_v], rows_v, sem).wait()
        pltpu.sync_copy(rows_v, out_hbm.at[pl.ds(base, b_per_w)])

    return gather_k


def _ce_kernel(n_valid, n_rows, x_ref, f_ref, g_ref, out_ref,
               xn_ref, s_ref):
    c = pl.program_id(0)
    nc = pl.num_programs(0)

    @pl.when(c == 0)
    def _init():
        x = x_ref[...]
        norm = jnp.sqrt(jnp.sum(x * x, axis=1, keepdims=True))
        scale = _LOG2E / (jnp.maximum(norm, 1e-12) * _TEMP)
        xn_ref[...] = (x * scale).astype(jnp.bfloat16)
        s_ref[...] = jnp.zeros_like(s_ref)

    xn = xn_ref[...]
    evs = []
    for k in range(_NSPLIT):
        m = jax.lax.dot_general(
            xn, f_ref[k * _SUB:(k + 1) * _SUB, :], (((1,), (1,)), ((), ())),
            preferred_element_type=jnp.float32)
        evs.append(jnp.exp2(m))

    @pl.when(c < nc - 1)
    def _mid():
        t = evs[0]
        for k in range(1, _NSPLIT):
            t = t + evs[k]
        s_ref[...] += t

    @pl.when(c == nc - 1)
    def _tail():
        lane = jax.lax.broadcasted_iota(jnp.int32, evs[0].shape, 1)
        acc = jnp.zeros_like(evs[0])
        for k in range(_NSPLIT):
            acc += jnp.where(lane < n_valid - c * _CHUNK - k * _SUB,
                             evs[k], 0.0)
        s_ref[...] += acc

    @pl.when(c == nc - 1)
    def _fin():
        lse = jnp.log(jnp.sum(s_ref[...], axis=1, keepdims=True))
        x = x_ref[...]
        norm = jnp.sqrt(jnp.sum(x * x, axis=1, keepdims=True))
        xh = x / (jnp.maximum(norm, 1e-12) * _TEMP)
        tl = jnp.sum(xh * g_ref[:, 0:64])
        out_ref[...] = ((jnp.sum(lse) - tl) * (1.0 / n_rows)).reshape(1, 1)


@jax.jit
def kernel(inputs, targets, cameras, features):
    b, d = inputs.shape
    n = features.shape[0]
    nc = pl.cdiv(n, _CHUNK)
    n_pad = nc * _CHUNK
    fpad = jnp.pad(features.astype(jnp.bfloat16), ((0, n_pad - n), (0, 0)))
    f128 = jnp.pad(features, ((0, 0), (0, 128 - d)))
    g = _make_sc_gather(b, 128)(f128, targets.astype(jnp.int32))
    out = pl.pallas_call(
        functools.partial(_ce_kernel, n, b),
        grid=(nc,),
        in_specs=[
            pl.BlockSpec((b, d), lambda i: (0, 0)),
            pl.BlockSpec((_CHUNK, d), lambda i: (i, 0)),
            pl.BlockSpec((b, 128), lambda i: (0, 0)),
        ],
        out_specs=pl.BlockSpec((1, 1), lambda i: (0, 0)),
        out_shape=jax.ShapeDtypeStruct((1, 1), jnp.float32),
        scratch_shapes=[
            pltpu.VMEM((b, d), jnp.bfloat16),
            pltpu.VMEM((b, 128), jnp.float32),
        ],
        compiler_params=pltpu.CompilerParams(
            dimension_semantics=("arbitrary",)),
    )(inputs, fpad, g)
    return out[0, 0]
